# baseline (speedup 1.0000x reference)
import functools

import jax
import jax.numpy as jnp
from jax import lax
from jax.experimental import pallas as pl
from jax.experimental.pallas import tpu as pltpu
from jax.experimental.pallas import tpu_sc as plsc

_N = 16384
_C = 1000
_NF = int(0.2 * _N)
_KEEP = _N - _NF
_BR = 512

_N_SC = 8192
_N_TC = _N - _N_SC
_NW = 32
_RPW = _N_SC // _NW
_BLK = 16

_R = 128
_S = _N // _R


def _loss_kernel(p1_ref, p2_ref, t_ref, l1_ref, l2_ref):
    t = t_ref[...]
    sel = jax.lax.broadcasted_iota(jnp.int32, (_BR, _C), 1) == t
    x1 = p1_ref[...]
    m1 = jnp.max(x1, axis=1, keepdims=True)
    s1 = jnp.sum(jnp.exp(x1 - m1), axis=1)
    xt1 = jnp.sum(jnp.where(sel, x1, 0.0), axis=1)
    l1_ref[...] = m1[:, 0] + jnp.log(s1) - xt1
    x2 = p2_ref[...]
    m2 = jnp.max(x2, axis=1, keepdims=True)
    s2 = jnp.sum(jnp.exp(x2 - m2), axis=1)
    xt2 = jnp.sum(jnp.where(sel, x2, 0.0), axis=1)
    l2_ref[...] = m2[:, 0] + jnp.log(s2) - xt2


def _sc_kernel(p1_hbm, p2_hbm, t_hbm, se1_hbm, se2_hbm, xt1_hbm, xt2_hbm,
               b1a, b2a, b1b, b2b, tgt_v, se1_v, se2_v, xt1_v, xt2_v,
               sem_a, sem_b):
    wid = lax.axis_index("s") * 2 + lax.axis_index("c")
    base = _N_TC + wid * _RPW
    pltpu.sync_copy(t_hbm.at[pl.ds(base, _RPW)], tgt_v)
    lanes = lax.iota(jnp.int32, 16)
    row_off = lanes * _C
    nblk = _RPW // _BLK
    bufs = ((b1a, b2a, sem_a), (b1b, b2b, sem_b))

    def issue(b, bset):
        r0 = base + b * _BLK
        c1 = pltpu.async_copy(p1_hbm.at[pl.ds(r0, _BLK)], bset[0], bset[2])
        c2 = pltpu.async_copy(p2_hbm.at[pl.ds(r0, _BLK)], bset[1], bset[2])
        return c1, c2

    pending = issue(0, bufs[0])
    for b in range(nblk):
        buf1, buf2, _ = bufs[b % 2]
        nxt = issue(b + 1, bufs[(b + 1) % 2]) if b + 1 < nblk else None
        pending[0].wait()
        pending[1].wait()
        zero = jnp.zeros((16,), jnp.float32)
        accs = (zero, zero, zero, zero, zero, zero, zero, zero)

        def col(cb, acc, buf1=buf1, buf2=buf2):
            a = list(acc)
            for u in range(8):
                ci = jnp.full((16,), cb * 8 + u, jnp.int32)
                g1 = plsc.load_gather(buf1, [lanes, ci])
                g2 = plsc.load_gather(buf2, [lanes, ci])
                a[u % 4] = a[u % 4] + jnp.exp(g1)
                a[4 + u % 4] = a[4 + u % 4] + jnp.exp(g2)
            return tuple(a)

        accs = lax.fori_loop(0, _C // 8, col, accs)
        se1 = (accs[0] + accs[1]) + (accs[2] + accs[3])
        se2 = (accs[4] + accs[5]) + (accs[6] + accs[7])
        tv = plsc.load_gather(tgt_v, [b * _BLK + lanes])
        g1 = plsc.load_gather(buf1, [lanes, tv])
        g2 = plsc.load_gather(buf2, [lanes, tv])
        out_idx = b * _BLK + lanes
        plsc.store_scatter(se1_v, [out_idx], se1)
        plsc.store_scatter(se2_v, [out_idx], se2)
        plsc.store_scatter(xt1_v, [out_idx], g1)
        plsc.store_scatter(xt2_v, [out_idx], g2)
        pending = nxt

    pltpu.sync_copy(se1_v, se1_hbm.at[pl.ds(wid * _RPW, _RPW)])
    pltpu.sync_copy(se2_v, se2_hbm.at[pl.ds(wid * _RPW, _RPW)])
    pltpu.sync_copy(xt1_v, xt1_hbm.at[pl.ds(wid * _RPW, _RPW)])
    pltpu.sync_copy(xt2_v, xt2_hbm.at[pl.ds(wid * _RPW, _RPW)])


def _select_kernel(a_tc_ref, b_tc_ref, se1_ref, se2_ref, xt1_ref, xt2_ref,
                   o1_ref, o2_ref):
    a_sc = jnp.maximum(jnp.log(se1_ref[...]) - xt1_ref[...], 0.0)
    b_sc = jnp.maximum(jnp.log(se2_ref[...]) - xt2_ref[...], 0.0)
    a = jnp.concatenate([a_tc_ref[...], a_sc], axis=0)
    b = jnp.concatenate([b_tc_ref[...], b_sc], axis=0)
    abits = jax.lax.bitcast_convert_type(a, jnp.int32)
    bbits = jax.lax.bitcast_convert_type(b, jnp.int32)

    def find_t(bits):
        def body(_, c):
            lo, hi = c
            mid = lo + ((hi - lo) >> 1)
            cnt = jnp.sum((bits <= mid).astype(jnp.int32), keepdims=True)
            ge = cnt >= _NF + 1
            return (jnp.where(ge, lo, mid + 1), jnp.where(ge, mid, hi))
        lo, _ = jax.lax.fori_loop(
            0, 31, body, (jnp.zeros((1, 1), jnp.int32),
                          jnp.full((1, 1), 0x7F800000, jnp.int32)))
        return lo

    ta = find_t(abits)
    tb = find_t(bbits)
    idx = (jax.lax.broadcasted_iota(jnp.int32, (_R, _S), 0) * _S
           + jax.lax.broadcasted_iota(jnp.int32, (_R, _S), 1))

    def keep_mask(bits, t):
        gt = bits > t
        eq = bits == t
        need = _KEEP - jnp.sum(gt.astype(jnp.int32), keepdims=True)
        def body(_, c):
            lo, hi = c
            mid = lo + ((hi - lo) >> 1)
            cnt = jnp.sum((eq & (idx >= mid)).astype(jnp.int32), keepdims=True)
            le = cnt <= need
            return (jnp.where(le, lo, mid + 1), jnp.where(le, mid, hi))
        m, _ = jax.lax.fori_loop(
            0, 15, body, (jnp.zeros((1, 1), jnp.int32),
                          jnp.full((1, 1), _N, jnp.int32)))
        return gt | (eq & (idx >= m))

    ka = keep_mask(abits, ta)
    kb = keep_mask(bbits, tb)
    o1_ref[0, 0] = jnp.sum(jnp.where(kb, a, 0.0)) / _KEEP
    o2_ref[0, 0] = jnp.sum(jnp.where(ka, b, 0.0)) / _KEEP


def kernel(pred1, pred2, target):
    t32 = target.astype(jnp.int32)

    sc_fn = pl.kernel(
        _sc_kernel,
        mesh=plsc.VectorSubcoreMesh(core_axis_name="c", subcore_axis_name="s"),
        compiler_params=pltpu.CompilerParams(needs_layout_passes=False),
        out_type=[jax.ShapeDtypeStruct((_N_SC,), jnp.float32)] * 4,
        scratch_types=[
            pltpu.VMEM((_BLK, _C), jnp.float32),
            pltpu.VMEM((_BLK, _C), jnp.float32),
            pltpu.VMEM((_BLK, _C), jnp.float32),
            pltpu.VMEM((_BLK, _C), jnp.float32),
            pltpu.VMEM((_RPW,), jnp.int32),
            pltpu.VMEM((_RPW,), jnp.float32),
            pltpu.VMEM((_RPW,), jnp.float32),
            pltpu.VMEM((_RPW,), jnp.float32),
            pltpu.VMEM((_RPW,), jnp.float32),
            pltpu.SemaphoreType.DMA,
            pltpu.SemaphoreType.DMA,
        ],
    )
    se1, se2, xt1, xt2 = sc_fn(pred1, pred2, t32)

    l1, l2 = pl.pallas_call(
        _loss_kernel,
        grid=(_N_TC // _BR,),
        in_specs=[pl.BlockSpec((_BR, _C), lambda i: (i, 0)),
                  pl.BlockSpec((_BR, _C), lambda i: (i, 0)),
                  pl.BlockSpec((_BR, 1), lambda i: (i, 0))],
        out_specs=[pl.BlockSpec((_BR,), lambda i: (i,)),
                   pl.BlockSpec((_BR,), lambda i: (i,))],
        out_shape=[jax.ShapeDtypeStruct((_N_TC,), jnp.float32)] * 2,
        compiler_params=pltpu.CompilerParams(
            dimension_semantics=("parallel",)),
    )(pred1, pred2, t32.reshape(_N, 1))

    o1, o2 = pl.pallas_call(
        _select_kernel,
        out_specs=[pl.BlockSpec(memory_space=pltpu.SMEM)] * 2,
        out_shape=[jax.ShapeDtypeStruct((1, 1), jnp.float32)] * 2,
    )(l1.reshape(_N_TC // _S, _S), l2.reshape(_N_TC // _S, _S),
      se1.reshape(_N_SC // _S, _S), se2.reshape(_N_SC // _S, _S),
      xt1.reshape(_N_SC // _S, _S), xt2.reshape(_N_SC // _S, _S))
    return (o1[0, 0], o2[0, 0])


# speedup vs baseline: 2.0416x; 2.0416x over previous
import jax
import jax.numpy as jnp
from jax.experimental import pallas as pl
from jax.experimental.pallas import tpu as pltpu

_N = 16384
_C = 1000
_NF = int(0.2 * _N)
_KEEP = _N - _NF
_BR = 512
_G = _N // _BR


def _row_losses(x, sel):
    m = jnp.max(x, axis=1, keepdims=True)
    s = jnp.sum(jnp.exp(x - m), axis=1)
    xt = jnp.sum(jnp.where(sel, x, 0.0), axis=1)
    return m[:, 0] + jnp.log(s) - xt


def _select(a, b, o1_ref, o2_ref):
    abits = jax.lax.bitcast_convert_type(a, jnp.int32)
    bbits = jax.lax.bitcast_convert_type(b, jnp.int32)

    def find_t(bits):
        def body(_, c):
            lo, hi = c
            mid = lo + ((hi - lo) >> 1)
            cnt = jnp.sum((bits <= mid).astype(jnp.int32), keepdims=True)
            ge = cnt >= _NF + 1
            return (jnp.where(ge, lo, mid + 1), jnp.where(ge, mid, hi))
        lo, _ = jax.lax.fori_loop(
            0, 31, body, (jnp.zeros((1, 1), jnp.int32),
                          jnp.full((1, 1), 0x7F800000, jnp.int32)))
        return lo

    ta = find_t(abits)
    tb = find_t(bbits)
    idx = (jax.lax.broadcasted_iota(jnp.int32, (_G, _BR), 0) * _BR
           + jax.lax.broadcasted_iota(jnp.int32, (_G, _BR), 1))

    def keep_mask(bits, t):
        gt = bits > t
        eq = bits == t
        need = _KEEP - jnp.sum(gt.astype(jnp.int32), keepdims=True)
        def body(_, c):
            lo, hi = c
            mid = lo + ((hi - lo) >> 1)
            cnt = jnp.sum((eq & (idx >= mid)).astype(jnp.int32), keepdims=True)
            le = cnt <= need
            return (jnp.where(le, lo, mid + 1), jnp.where(le, mid, hi))
        m, _ = jax.lax.fori_loop(
            0, 15, body, (jnp.zeros((1, 1), jnp.int32),
                          jnp.full((1, 1), _N, jnp.int32)))
        return gt | (eq & (idx >= m))

    ka = keep_mask(abits, ta)
    kb = keep_mask(bbits, tb)
    o1_ref[0, 0] = jnp.sum(jnp.where(kb, a, 0.0)) / _KEEP
    o2_ref[0, 0] = jnp.sum(jnp.where(ka, b, 0.0)) / _KEEP


def _kernel(p1_ref, p2_ref, t_ref, o1_ref, o2_ref, l1_s, l2_s):
    i = pl.program_id(0)
    t = t_ref[...]
    sel = jax.lax.broadcasted_iota(jnp.int32, (_BR, _C), 1) == t
    l1_s[pl.ds(i, 1), :] = _row_losses(p1_ref[...], sel).reshape(1, _BR)
    l2_s[pl.ds(i, 1), :] = _row_losses(p2_ref[...], sel).reshape(1, _BR)

    @pl.when(i == _G - 1)
    def _():
        _select(l1_s[...], l2_s[...], o1_ref, o2_ref)


def kernel(pred1, pred2, target):
    t = target.astype(jnp.int32).reshape(_N, 1)
    o1, o2 = pl.pallas_call(
        _kernel,
        grid=(_G,),
        in_specs=[pl.BlockSpec((_BR, _C), lambda i: (i, 0)),
                  pl.BlockSpec((_BR, _C), lambda i: (i, 0)),
                  pl.BlockSpec((_BR, 1), lambda i: (i, 0))],
        out_specs=[pl.BlockSpec(memory_space=pltpu.SMEM)] * 2,
        out_shape=[jax.ShapeDtypeStruct((1, 1), jnp.float32)] * 2,
        scratch_shapes=[pltpu.VMEM((_G, _BR), jnp.float32)] * 2,
    )(pred1, pred2, t)
    return (o1[0, 0], o2[0, 0])
